# baseline (speedup 1.0000x reference)
import functools

import jax
import jax.numpy as jnp
from jax import lax
from jax.experimental import pallas as pl
from jax.experimental.pallas import tpu as pltpu
from jax.experimental.pallas import tpu_sc as plsc

NC = 9
_STD = 0.1
_AVG = 1.0

_B = 16
_PIX = 512 * 512
_NW = 32
_HALF = _PIX // 2
_CH = 32768
_GROUPS = _CH // (8 * 16)


_SC_ROWS = 256
_TC_ROWS = 512 - _SC_ROWS
_ROWS = 64
_NCHUNK = (_SC_ROWS // 2) // _ROWS


def _sc_hist_body(labels_hbm, out_hbm, buf0, buf1, cnt_ref, sem0, sem1):
    wid = lax.axis_index("s") * 2 + lax.axis_index("c")
    b = wid >> 1
    row0 = (wid & 1) * (_SC_ROWS // 2)

    bufs = (buf0, buf1)
    sems = (sem0, sem1)

    def start(k):
        return pltpu.async_copy(
            labels_hbm.at[b, pl.ds(row0 + k * _ROWS, _ROWS), :],
            bufs[k % 2], sems[k % 2])

    cp = start(0)
    accs = tuple(jnp.zeros((16,), jnp.int32) for _ in range(8))
    for k in range(_NCHUNK):
        nxt = start(k + 1) if k + 1 < _NCHUNK else None
        cp.wait()
        buf = bufs[k % 2]

        def body(r, accs):
            pcs = [jnp.zeros((16,), jnp.int32) for _ in range(4)]
            sss = [jnp.zeros((16,), jnp.int32) for _ in range(2)]
            for j in range(32):
                lab = buf[r, pl.ds(pl.multiple_of(j * 16, 16), 16)]
                pcs[j & 3] = pcs[j & 3] + jnp.left_shift(1, lab << 2)
                sss[j & 1] = sss[j & 1] + lab
            pc = (pcs[0] + pcs[1]) + (pcs[2] + pcs[3])
            out = [accs[0] + sss[0] + sss[1]]
            out.extend(accs[c] + ((pc >> (4 * c)) & 15) for c in range(1, 8))
            return tuple(out)

        accs = lax.fori_loop(0, _ROWS, body, accs)
        cp = nxt

    for c in range(8):
        cnt_ref[c, :] = accs[c]
    pltpu.sync_copy(cnt_ref, out_hbm.at[wid])


def _sc_hist(labels):
    mesh = plsc.VectorSubcoreMesh(core_axis_name="c", subcore_axis_name="s")
    run = functools.partial(
        pl.kernel,
        mesh=mesh,
        out_type=jax.ShapeDtypeStruct((_NW, 8, 16), jnp.int32),
        scratch_types=[
            pltpu.VMEM((_ROWS, 512), jnp.int32),
            pltpu.VMEM((_ROWS, 512), jnp.int32),
            pltpu.VMEM((8, 16), jnp.int32),
            pltpu.SemaphoreType.DMA,
            pltpu.SemaphoreType.DMA,
        ],
    )(_sc_hist_body)
    return run(labels)


def _tc_hist_kernel(lab_ref, out_ref):
    b = pl.program_id(0)
    x = lab_ref[0]
    pc = jnp.left_shift(1, x << 2)
    for _ in range(3):
        n = pc.shape[0] // 2
        pc = pc[:n] + pc[n:]
    out_ref[b, 0] = jnp.sum(x)
    for c in range(1, 8):
        out_ref[b, c] = jnp.sum((pc >> (4 * c)) & 15)


def _mul_kernel(cnt_ref, cnt2_ref, preds_ref, out_ref, w_ref):
    nb = pl.program_id(1)

    @pl.when(nb == 0)
    def _():
        acc = cnt_ref[0].astype(jnp.float32)
        per = jnp.sum(acc, axis=(0, 2))
        bi = pl.program_id(0)
        n17 = [per[c] + cnt2_ref[bi, c].astype(jnp.float32)
               for c in range(1, 8)]
        s_tot = per[0] + cnt2_ref[bi, 0].astype(jnp.float32)
        c8 = (s_tot - sum(c * n for c, n in enumerate(n17, start=1))) / 8.0
        hist = [_PIX - sum(n17) - c8] + n17 + [c8]

        h = [jnp.where(hc > 0.0, -jnp.log(hc / _PIX), 0.0) for hc in hist]
        cnt = sum(jnp.where(hc > 0.0, 1.0, 0.0) for hc in hist)
        mean = sum(h) / cnt
        var = sum(jnp.where(hc > 0.0, (hh - mean) ** 2, 0.0)
                  for hc, hh in zip(hist, h)) / cnt
        std = jnp.sqrt(var)
        for c in range(NC):
            w_ref[c] = jnp.where(h[c] != 0.0,
                                 (h[c] - mean) / std * _STD + _AVG, 1.0)

    for c in range(NC):
        out_ref[0, c] = preds_ref[0, c] * w_ref[c]


def kernel(preds, labels):
    B, C, H, W = preds.shape
    counts = _sc_hist(labels).reshape(B, 2, 8, 16)
    counts2 = pl.pallas_call(
        _tc_hist_kernel,
        grid=(B,),
        in_specs=[
            pl.BlockSpec((1, _TC_ROWS, W),
                         lambda b: (b, _SC_ROWS // _TC_ROWS, 0)),
        ],
        out_specs=pl.BlockSpec((B, 8), lambda b: (0, 0),
                               memory_space=pltpu.SMEM),
        out_shape=jax.ShapeDtypeStruct((B, 8), jnp.int32),
    )(labels)
    NB = 1
    HB = H // NB
    return pl.pallas_call(
        _mul_kernel,
        grid=(B, NB),
        in_specs=[
            pl.BlockSpec((1, 2, 8, 16), lambda b, nb: (b, 0, 0, 0)),
            pl.BlockSpec((B, 8), lambda b, nb: (0, 0),
                         memory_space=pltpu.SMEM),
            pl.BlockSpec((1, C, HB, W), lambda b, nb: (b, 0, nb, 0)),
        ],
        out_specs=pl.BlockSpec((1, C, HB, W), lambda b, nb: (b, 0, nb, 0)),
        out_shape=jax.ShapeDtypeStruct((B, C, H, W), preds.dtype),
        scratch_shapes=[pltpu.SMEM((NC,), jnp.float32)],
        compiler_params=pltpu.CompilerParams(
            dimension_semantics=("parallel", "arbitrary")),
    )(counts, counts2, preds)


# speedup vs baseline: 1.0119x; 1.0119x over previous
import functools

import jax
import jax.numpy as jnp
from jax import lax
from jax.experimental import pallas as pl
from jax.experimental.pallas import tpu as pltpu
from jax.experimental.pallas import tpu_sc as plsc

NC = 9
_STD = 0.1
_AVG = 1.0

_B = 16
_PIX = 512 * 512
_NW = 32
_HALF = _PIX // 2
_CH = 32768
_GROUPS = _CH // (8 * 16)


_SC_ROWS = 256
_TC_ROWS = 512 - _SC_ROWS
_TC_BLK = 8
_ROWS = 64
_NCHUNK = (_SC_ROWS // 2) // _ROWS


def _sc_hist_body(labels_hbm, out_hbm, buf0, buf1, cnt_ref, sem0, sem1):
    wid = lax.axis_index("s") * 2 + lax.axis_index("c")
    b = wid >> 1
    row0 = (wid & 1) * (_SC_ROWS // 2)

    bufs = (buf0, buf1)
    sems = (sem0, sem1)

    def start(k):
        return pltpu.async_copy(
            labels_hbm.at[b, pl.ds(row0 + k * _ROWS, _ROWS), :],
            bufs[k % 2], sems[k % 2])

    cp = start(0)
    accs = tuple(jnp.zeros((16,), jnp.int32) for _ in range(8))
    for k in range(_NCHUNK):
        nxt = start(k + 1) if k + 1 < _NCHUNK else None
        cp.wait()
        buf = bufs[k % 2]

        def body(r, accs):
            pcs = [jnp.zeros((16,), jnp.int32) for _ in range(4)]
            sss = [jnp.zeros((16,), jnp.int32) for _ in range(2)]
            for j in range(32):
                lab = buf[r, pl.ds(pl.multiple_of(j * 16, 16), 16)]
                pcs[j & 3] = pcs[j & 3] + jnp.left_shift(1, lab << 2)
                sss[j & 1] = sss[j & 1] + lab
            pc = (pcs[0] + pcs[1]) + (pcs[2] + pcs[3])
            out = [accs[0] + sss[0] + sss[1]]
            out.extend(accs[c] + ((pc >> (4 * c)) & 15) for c in range(1, 8))
            return tuple(out)

        accs = lax.fori_loop(0, _ROWS, body, accs)
        cp = nxt

    for c in range(8):
        cnt_ref[c, :] = accs[c]
    pltpu.sync_copy(cnt_ref, out_hbm.at[wid])


def _sc_hist(labels):
    mesh = plsc.VectorSubcoreMesh(core_axis_name="c", subcore_axis_name="s")
    run = functools.partial(
        pl.kernel,
        mesh=mesh,
        out_type=jax.ShapeDtypeStruct((_NW, 8, 16), jnp.int32),
        scratch_types=[
            pltpu.VMEM((_ROWS, 512), jnp.int32),
            pltpu.VMEM((_ROWS, 512), jnp.int32),
            pltpu.VMEM((8, 16), jnp.int32),
            pltpu.SemaphoreType.DMA,
            pltpu.SemaphoreType.DMA,
        ],
    )(_sc_hist_body)
    return run(labels)


def _tc_hist_kernel(lab_ref, out_ref):
    for i in range(_TC_BLK):
        x = lab_ref[i]
        pc = jnp.left_shift(1, x << 2)
        for _ in range(3):
            n = pc.shape[0] // 2
            pc = pc[:n] + pc[n:]
        out_ref[i, 0] = jnp.sum(x)
        for c in range(1, 8):
            out_ref[i, c] = jnp.sum((pc >> (4 * c)) & 15)


def _mul_kernel(cnt_ref, cnt2_ref, preds_ref, out_ref, w_ref):
    nb = pl.program_id(1)

    @pl.when(nb == 0)
    def _():
        acc = cnt_ref[0].astype(jnp.float32)
        per = jnp.sum(acc, axis=(0, 2))
        bi = pl.program_id(0)
        n17 = [per[c] + cnt2_ref[bi, c].astype(jnp.float32)
               for c in range(1, 8)]
        s_tot = per[0] + cnt2_ref[bi, 0].astype(jnp.float32)
        c8 = (s_tot - sum(c * n for c, n in enumerate(n17, start=1))) / 8.0
        hist = [_PIX - sum(n17) - c8] + n17 + [c8]

        h = [jnp.where(hc > 0.0, -jnp.log(hc / _PIX), 0.0) for hc in hist]
        cnt = sum(jnp.where(hc > 0.0, 1.0, 0.0) for hc in hist)
        mean = sum(h) / cnt
        var = sum(jnp.where(hc > 0.0, (hh - mean) ** 2, 0.0)
                  for hc, hh in zip(hist, h)) / cnt
        std = jnp.sqrt(var)
        for c in range(NC):
            w_ref[c] = jnp.where(h[c] != 0.0,
                                 (h[c] - mean) / std * _STD + _AVG, 1.0)

    for c in range(NC):
        out_ref[0, c] = preds_ref[0, c] * w_ref[c]


def kernel(preds, labels):
    B, C, H, W = preds.shape
    counts = _sc_hist(labels).reshape(B, 2, 8, 16)
    counts2 = pl.pallas_call(
        _tc_hist_kernel,
        grid=(B // _TC_BLK,),
        in_specs=[
            pl.BlockSpec((_TC_BLK, _TC_ROWS, W),
                         lambda s: (s, _SC_ROWS // _TC_ROWS, 0)),
        ],
        out_specs=pl.BlockSpec((_TC_BLK, 8), lambda s: (s, 0),
                               memory_space=pltpu.SMEM),
        out_shape=jax.ShapeDtypeStruct((B, 8), jnp.int32),
    )(labels)
    NB = 1
    HB = H // NB
    return pl.pallas_call(
        _mul_kernel,
        grid=(B, NB),
        in_specs=[
            pl.BlockSpec((1, 2, 8, 16), lambda b, nb: (b, 0, 0, 0)),
            pl.BlockSpec((B, 8), lambda b, nb: (0, 0),
                         memory_space=pltpu.SMEM),
            pl.BlockSpec((1, C, HB, W), lambda b, nb: (b, 0, nb, 0)),
        ],
        out_specs=pl.BlockSpec((1, C, HB, W), lambda b, nb: (b, 0, nb, 0)),
        out_shape=jax.ShapeDtypeStruct((B, C, H, W), preds.dtype),
        scratch_shapes=[pltpu.SMEM((NC,), jnp.float32)],
        compiler_params=pltpu.CompilerParams(
            dimension_semantics=("parallel", "arbitrary")),
    )(counts, counts2, preds)
